# baseline (speedup 1.0000x reference)
import functools

import jax
import jax.numpy as jnp
from jax import lax
from jax.experimental import pallas as pl
from jax.experimental.pallas import tpu as pltpu
from jax.experimental.pallas import tpu_sc as plsc

N_NODES = 10000
N_PAD = 10240
N_EDGES = 160000
DIM = 3
KS = 25
S = 8
KC = KS ** DIM
EPS = 1e-5

TILE = 16
CT = 128
CHUNK_P = CT * TILE
NP = N_EDGES * S
NTILE = 94720
NCH = NTILE // CT
P_CAP = NTILE * TILE
G = 128
NW = 32
PW = P_CAP // NW
RING = 4
PF = 2

_USE_SC_GATHER = True
_USE_SC_SCATTER = True

assert P_CAP % (NW * G) == 0
assert NP + KC * (TILE - 1) <= P_CAP


def _rup(a, b):
    return (a + b - 1) // b * b


def _basis_body(ea, bas, kid):
    u = jnp.clip(ea[...], 0.0, 1.0)
    v = u * (KS - 1)
    lo = jnp.floor(v)
    frac = v - lo
    loi = lo.astype(jnp.int32)
    for s in range(S):
        w = None
        idx = None
        stride = 1
        for d in range(DIM):
            bit = (s >> d) & 1
            fd = frac[d:d + 1, :]
            wd = fd if bit == 1 else (1.0 - fd)
            w = wd if w is None else w * wd
            id_d = jnp.clip(loi[d:d + 1, :] + bit, 0, KS - 1) * stride
            idx = id_d if idx is None else idx + id_d
            stride *= KS
        bas[s:s + 1, :] = w
        kid[s:s + 1, :] = idx


def _spline_basis(ea_t):
    BL = 3200
    grid = (N_EDGES // BL,)
    return pl.pallas_call(
        _basis_body,
        grid=grid,
        in_specs=[pl.BlockSpec((DIM, BL), lambda c: (0, c))],
        out_specs=[
            pl.BlockSpec((S, BL), lambda c: (0, c)),
            pl.BlockSpec((S, BL), lambda c: (0, c)),
        ],
        out_shape=[
            jax.ShapeDtypeStruct((S, N_EDGES), jnp.float32),
            jax.ShapeDtypeStruct((S, N_EDGES), jnp.int32),
        ],
    )(ea_t)


def _routing(flat_k, flat_b, src, dst):
    pairidx = jnp.arange(NP, dtype=jnp.int32)
    sk, sp = lax.sort((flat_k, pairidx), num_keys=1)
    se = sp % N_EDGES
    sb = jnp.take(flat_b, sp)
    ssrc = jnp.take(src, se)
    sdst = jnp.take(dst, se)

    counts = jnp.zeros((KC,), jnp.int32).at[flat_k].add(1)
    csum = jnp.cumsum(counts)
    off = csum - counts
    pc = (counts + (TILE - 1)) // TILE * TILE
    pcs = jnp.cumsum(pc)
    poff = pcs - pc

    j = jnp.arange(NP, dtype=jnp.int32)
    pos = jnp.take(poff, sk) + (j - jnp.take(off, sk))

    padded_src = jnp.zeros((P_CAP,), jnp.int32).at[pos].set(ssrc)
    padded_dst = jnp.zeros((P_CAP,), jnp.int32).at[pos].set(sdst)
    padded_bas = jnp.zeros((P_CAP,), jnp.float32).at[pos].set(sb)
    padded_k = jnp.zeros((P_CAP,), jnp.int32).at[pos].set(sk)
    tile_k = padded_k[::TILE]

    tt = jnp.arange(NTILE, dtype=jnp.int32)
    is_start = (tt % CT == 0) | (tile_k != jnp.roll(tile_k, 1))
    isi = is_start.astype(jnp.int32)
    rid = jnp.cumsum(isi) - 1
    nruns_c = jnp.zeros((NCH,), jnp.int32).at[tt // CT].add(isi)
    crs = jnp.cumsum(nruns_c) - nruns_c

    idxr = jnp.where(is_start, rid, NTILE)
    run_k = jnp.zeros((NTILE,), jnp.int32).at[idxr].set(tile_k, mode='drop')
    run_t0 = jnp.zeros((NTILE,), jnp.int32).at[idxr].set(tt, mode='drop')
    run_nt = jnp.zeros((NTILE,), jnp.int32).at[rid].add(1)

    cr = run_t0 // CT
    slot = jnp.arange(NTILE, dtype=jnp.int32) - jnp.take(crs, cr)
    tabk = jnp.zeros((NCH, CT), jnp.int32).at[cr, slot].set(run_k, mode='drop')
    tabt0 = jnp.zeros((NCH, CT), jnp.int32).at[cr, slot].set(
        run_t0 - cr * CT, mode='drop')
    tabnt = jnp.zeros((NCH, CT), jnp.int32).at[cr, slot].set(
        run_nt, mode='drop')

    bas3 = padded_bas.reshape(NCH, 1, CHUNK_P)
    return (padded_src, padded_dst, bas3,
            tabk.reshape(NCH, 1, CT), tabt0.reshape(NCH, 1, CT),
            tabnt.reshape(NCH, 1, CT), nruns_c.reshape(NCH, 1, 1))


def _sc_gather(h_pad, idx, ci_pad):
    mesh = plsc.VectorSubcoreMesh(core_axis_name="c", subcore_axis_name="s")

    @functools.partial(
        pl.kernel, mesh=mesh,
        out_type=jax.ShapeDtypeStruct((P_CAP, ci_pad), jnp.float32),
        scratch_types=[
            pltpu.VMEM((G,), jnp.int32),
            pltpu.VMEM((G, ci_pad), jnp.float32),
        ],
    )
    def k(h_hbm, idx_hbm, out_hbm, idx_v, rows_v):
        c = lax.axis_index("c")
        s = lax.axis_index("s")
        w = s * 2 + c
        base = w * PW

        @pl.loop(0, PW // G)
        def _(i):
            b = base + i * G
            pltpu.sync_copy(idx_hbm.at[pl.ds(b, G)], idx_v)
            pltpu.sync_copy(h_hbm.at[idx_v], rows_v)
            pltpu.sync_copy(rows_v, out_hbm.at[pl.ds(b, G)])

    return k(h_pad, idx)


def _sc_scatter(y, dst, zeros_hbm, co_pad):
    mesh = plsc.VectorSubcoreMesh(core_axis_name="c", subcore_axis_name="s")
    STRIPE = N_PAD // 16

    @functools.partial(
        pl.kernel, mesh=mesh,
        out_type=jax.ShapeDtypeStruct((2 * N_PAD, co_pad), jnp.float32),
        scratch_types=[
            pltpu.VMEM((G,), jnp.int32),
            pltpu.VMEM((G, co_pad), jnp.float32),
            pltpu.VMEM_SHARED((N_PAD, co_pad), jnp.float32),
        ],
    )
    def k(y_hbm, dst_hbm, z_hbm, out_hbm, idx_v, rows_v, acc):
        c = lax.axis_index("c")
        s = lax.axis_index("s")
        w = s * 2 + c

        pltpu.sync_copy(z_hbm, rows_v)

        @pl.loop(0, STRIPE // G)
        def _(i):
            pltpu.sync_copy(rows_v, acc.at[pl.ds(s * STRIPE + i * G, G)])

        plsc.subcore_barrier()
        base = w * PW

        @pl.loop(0, PW // G)
        def _(i):
            b = base + i * G
            pltpu.sync_copy(dst_hbm.at[pl.ds(b, G)], idx_v)
            pltpu.sync_copy(y_hbm.at[pl.ds(b, G)], rows_v)
            pltpu.sync_copy(rows_v, acc.at[idx_v], add=True)

        plsc.subcore_barrier()

        @pl.loop(0, STRIPE // G)
        def _(i):
            r = s * STRIPE + i * G
            pltpu.sync_copy(acc.at[pl.ds(r, G)], rows_v)
            pltpu.sync_copy(rows_v, out_hbm.at[pl.ds(c * N_PAD + r, G)])

    return k(y, dst, zeros_hbm).reshape(2, N_PAD, co_pad)


def _gmm_body(ci, co, ci_pad, co_pad,
              xg, bas, tabk, tabt0, tabnt, nrun, w_any, y,
              scaled, wring, sems):
    nr = nrun[0, 0, 0]
    y[...] = jnp.zeros((CHUNK_P, co_pad), jnp.float32)
    bcol = bas[0].reshape(CHUNK_P, 1)
    scaled[...] = xg[...] * bcol

    def issue(r):
        kk = tabk[0, 0, r]
        slot = lax.rem(r, RING)
        pltpu.make_async_copy(w_any.at[kk], wring.at[slot],
                              sems.at[slot]).start()

    for r0 in range(PF):
        @pl.when(r0 < nr)
        def _():
            issue(r0)

    def run_body(r, _):
        @pl.when(r + PF < nr)
        def _():
            issue(r + PF)
        slot = lax.rem(r, RING)
        pltpu.make_async_copy(w_any.at[tabk[0, 0, r]], wring.at[slot],
                              sems.at[slot]).wait()
        t0 = tabt0[0, 0, r]
        ntl = tabnt[0, 0, r]
        wmat = wring[slot]

        def tile_body(tt, _):
            row0 = pl.multiple_of((t0 + tt) * TILE, TILE)
            xb = scaled[pl.ds(row0, TILE), :ci]
            yv = jnp.dot(xb, wmat, preferred_element_type=jnp.float32)
            y[pl.ds(row0, TILE), :co] = yv
            return 0

        lax.fori_loop(0, ntl, tile_body, 0)
        return 0

    lax.fori_loop(0, nr, run_body, 0)


def _tc_groupmm(Xg, bas3, tabk, tabt0, tabnt, nruns, W, ci, co,
                ci_pad, co_pad):
    body = functools.partial(_gmm_body, ci, co, ci_pad, co_pad)
    return pl.pallas_call(
        body,
        grid=(NCH,),
        in_specs=[
            pl.BlockSpec((CHUNK_P, ci_pad), lambda c: (c, 0)),
            pl.BlockSpec((1, 1, CHUNK_P), lambda c: (c, 0, 0)),
            pl.BlockSpec((1, 1, CT), lambda c: (c, 0, 0),
                         memory_space=pltpu.SMEM),
            pl.BlockSpec((1, 1, CT), lambda c: (c, 0, 0),
                         memory_space=pltpu.SMEM),
            pl.BlockSpec((1, 1, CT), lambda c: (c, 0, 0),
                         memory_space=pltpu.SMEM),
            pl.BlockSpec((1, 1, 1), lambda c: (c, 0, 0),
                         memory_space=pltpu.SMEM),
            pl.BlockSpec(memory_space=pl.ANY),
        ],
        out_specs=pl.BlockSpec((CHUNK_P, co_pad), lambda c: (c, 0)),
        out_shape=jax.ShapeDtypeStruct((P_CAP, co_pad), jnp.float32),
        scratch_shapes=[
            pltpu.VMEM((CHUNK_P, ci_pad), jnp.float32),
            pltpu.VMEM((RING, ci, co), jnp.float32),
            pltpu.SemaphoreType.DMA((RING,)),
        ],
    )(Xg, bas3, tabk, tabt0, tabnt, nruns, W)


def _dense_body(ci, co, co_pad, elu, parts, h, root, a, b, out):
    q = parts[0] + parts[1]
    xr = jnp.dot(h[:, :ci], root[...], preferred_element_type=jnp.float32)
    if co_pad > co:
        xr = jnp.concatenate(
            [xr, jnp.zeros((xr.shape[0], co_pad - co), jnp.float32)], axis=1)
    val = (q + xr) * a[...] + b[...]
    if elu:
        val = jnp.where(val > 0, val, jnp.exp(jnp.minimum(val, 0.0)) - 1.0)
    out[...] = val


def _tc_dense(parts, h, root, a2, b2, ci, ci_pad, co, co_pad, elu):
    NB = 2048
    body = functools.partial(_dense_body, ci, co, co_pad, elu)
    return pl.pallas_call(
        body,
        grid=(N_PAD // NB,),
        in_specs=[
            pl.BlockSpec((2, NB, co_pad), lambda c: (0, c, 0)),
            pl.BlockSpec((NB, ci_pad), lambda c: (c, 0)),
            pl.BlockSpec((ci, co), lambda c: (0, 0)),
            pl.BlockSpec((1, co_pad), lambda c: (0, 0)),
            pl.BlockSpec((1, co_pad), lambda c: (0, 0)),
        ],
        out_specs=pl.BlockSpec((NB, co_pad), lambda c: (c, 0)),
        out_shape=jax.ShapeDtypeStruct((N_PAD, co_pad), jnp.float32),
    )(parts, h, root, a2, b2)


def kernel(x, edge_index, edge_attr,
           W1, root1, b1, g1, be1, rm1, rv1,
           W2, root2, b2, g2, be2, rm2, rv2,
           W3, root3, b3, g3, be3, rm3, rv3,
           W4, root4, b4):
    src = edge_index[0]
    dst = edge_index[1]

    bas2, kid2 = _spline_basis(edge_attr.T)
    (padded_src, padded_dst, bas3, tabk, tabt0, tabnt, nruns) = _routing(
        kid2.reshape(-1), bas2.reshape(-1), src, dst)

    layers = []
    for (W, root, bias, bn) in (
            (W1, root1, b1, (g1, be1, rm1, rv1)),
            (W2, root2, b2, (g2, be2, rm2, rv2)),
            (W3, root3, b3, (g3, be3, rm3, rv3)),
            (W4, root4, b4, None)):
        ci, co = root.shape
        ci_pad = 128
        co_pad = 128
        if bn is not None:
            g, be, rm, rv = bn
            sc = g / jnp.sqrt(rv + EPS)
            a = sc
            b = (bias - rm) * sc + be
        else:
            a = jnp.ones((co,), jnp.float32)
            b = bias
        a2 = jnp.pad(a, (0, co_pad - co)).reshape(1, co_pad)
        b2 = jnp.pad(b, (0, co_pad - co)).reshape(1, co_pad)
        layers.append((W, root, a2, b2, ci, co, ci_pad, co_pad))

    h = jnp.pad(x, ((0, N_PAD - N_NODES), (0, layers[0][6] - x.shape[1])))
    for li, (W, root, a2, b2, ci, co, ci_pad, co_pad) in enumerate(layers):
        if _USE_SC_GATHER:
            Xg = _sc_gather(h, padded_src, ci_pad)
        else:
            Xg = jnp.take(h, padded_src, axis=0)
        Y = _tc_groupmm(Xg, bas3, tabk, tabt0, tabnt, nruns, W,
                        ci, co, ci_pad, co_pad)
        if _USE_SC_SCATTER:
            zeros_hbm = jnp.zeros((G, co_pad), jnp.float32)
            parts = _sc_scatter(Y, padded_dst, zeros_hbm, co_pad)
        else:
            agg = jax.ops.segment_sum(Y, padded_dst, num_segments=N_PAD)
            parts = jnp.stack([agg, jnp.zeros_like(agg)])
        h = _tc_dense(parts, h, root, a2, b2, ci, ci_pad, co, co_pad,
                      elu=(li < 3))
    return h[:N_NODES, :50]
